# Initial kernel scaffold; baseline (speedup 1.0000x reference)
import jax
import jax.numpy as jnp
from jax.experimental import pallas as pl


def kernel(X, table, bias):
    raise NotImplementedError("write your pallas kernel here")


# speedup vs baseline: 1.3188x; 1.3188x over previous
import functools

import jax
import jax.numpy as jnp
from jax import lax
from jax.experimental import pallas as pl
from jax.experimental.pallas import tpu as pltpu
from jax.experimental.pallas import tpu_sc as plsc

N_FIELDS_K = 26
EMBED_DIM_K = 32
BATCH_K = 16384

NUM_WORKERS = 32
TOTAL_ROWS = BATCH_K * N_FIELDS_K
ROWS_PER_WORKER = TOTAL_ROWS // NUM_WORKERS
GROUP = 104
GROUPS_PER_WORKER = ROWS_PER_WORKER // GROUP


def _sc_body(table_h, idx_h, bias_h, out_h, idx_v, bias_v, buf_v, sem):
    wid = lax.axis_index("s") * 2 + lax.axis_index("c")
    base = wid * ROWS_PER_WORKER

    pltpu.sync_copy(idx_h.at[wid], idx_v)
    pltpu.sync_copy(bias_h, bias_v)

    def group_step(g, _):
        pltpu.async_copy(table_h.at[idx_v.at[g]], buf_v, sem).wait()
        for r in range(GROUP):
            for c in range(EMBED_DIM_K // 16):
                sl = pl.ds(c * 16, 16)
                buf_v[r, sl] = buf_v[r, sl] + bias_v[r, sl]
        off = pl.multiple_of(base + g * GROUP, 8)
        pltpu.sync_copy(buf_v, out_h.at[pl.ds(off, GROUP)])
        return 0

    lax.fori_loop(0, GROUPS_PER_WORKER, group_step, 0)


@jax.jit
def kernel(X, table, bias):
    idx = X.reshape(NUM_WORKERS, GROUPS_PER_WORKER, GROUP)
    bias_tiled = jnp.tile(bias, (GROUP // N_FIELDS_K, 1))

    mesh = plsc.VectorSubcoreMesh(core_axis_name="c", subcore_axis_name="s")
    run = functools.partial(
        pl.kernel,
        mesh=mesh,
        out_type=jax.ShapeDtypeStruct((TOTAL_ROWS, EMBED_DIM_K), jnp.float32),
        scratch_types=[
            pltpu.VMEM((GROUPS_PER_WORKER, GROUP), jnp.int32),
            pltpu.VMEM((GROUP, EMBED_DIM_K), jnp.float32),
            pltpu.VMEM((GROUP, EMBED_DIM_K), jnp.float32),
            pltpu.SemaphoreType.DMA,
        ],
        compiler_params=pltpu.CompilerParams(use_tc_tiling_on_sc=False),
    )(_sc_body)
    out = run(table, idx, bias_tiled)
    return out.reshape(BATCH_K, N_FIELDS_K, EMBED_DIM_K)


# speedup vs baseline: 1.3884x; 1.0528x over previous
import functools

import jax
import jax.numpy as jnp
from jax import lax
from jax.experimental import pallas as pl
from jax.experimental.pallas import tpu as pltpu
from jax.experimental.pallas import tpu_sc as plsc

N_FIELDS_K = 26
EMBED_DIM_K = 32
BATCH_K = 16384

NUM_WORKERS = 32
TOTAL_ROWS = BATCH_K * N_FIELDS_K
ROWS_PER_WORKER = TOTAL_ROWS // NUM_WORKERS
GROUP = 104
GROUPS_PER_WORKER = ROWS_PER_WORKER // GROUP
K_GROUPS = 8
CHUNK_ROWS = K_GROUPS * GROUP
NUM_CHUNKS = GROUPS_PER_WORKER // K_GROUPS
NLANE = 16


def _sc_body(table_h, idx_h, bias_h, out_h,
             idx_v, bias_v, buf0, buf1, sg0, sg1, ss0, ss1):
    wid = lax.axis_index("s") * 2 + lax.axis_index("c")
    base = wid * ROWS_PER_WORKER

    pltpu.sync_copy(idx_h.at[wid], idx_v)
    pltpu.sync_copy(bias_h, bias_v)

    def fire(c, buf, sem):
        for j in range(K_GROUPS):
            pltpu.async_copy(
                table_h.at[idx_v.at[c * K_GROUPS + j]],
                buf.at[pl.ds(j * GROUP, GROUP)],
                sem,
            )

    def drain(buf, sem):
        pltpu.make_async_copy(out_h.at[pl.ds(0, CHUNK_ROWS)], buf, sem).wait()

    def add_bias(buf):
        def per_group(g, _):
            for r in range(GROUP):
                row = g * GROUP + r
                for c16 in range(EMBED_DIM_K // NLANE):
                    sl = pl.ds(c16 * NLANE, NLANE)
                    buf[row, sl] = buf[row, sl] + bias_v[r, sl]
            return 0
        lax.fori_loop(0, K_GROUPS, per_group, 0)

    def scatter(c, buf, sem):
        off = pl.multiple_of(base + c * CHUNK_ROWS, 8)
        pltpu.async_copy(buf, out_h.at[pl.ds(off, CHUNK_ROWS)], sem)

    def wait_scatter(c, buf, sem):
        off = pl.multiple_of(base + c * CHUNK_ROWS, 8)
        pltpu.make_async_copy(buf, out_h.at[pl.ds(off, CHUNK_ROWS)], sem).wait()

    fire(0, buf0, sg0)

    def pair_step(p, _):
        c0 = 2 * p
        c1 = c0 + 1
        @pl.when(p > 0)
        def _():
            wait_scatter(c1 - 2, buf1, ss1)
        fire(c1, buf1, sg1)
        drain(buf0, sg0)
        add_bias(buf0)
        scatter(c0, buf0, ss0)
        @pl.when(p < NUM_CHUNKS // 2 - 1)
        def _():
            wait_scatter(c0, buf0, ss0)
            fire(c0 + 2, buf0, sg0)
        drain(buf1, sg1)
        add_bias(buf1)
        scatter(c1, buf1, ss1)
        return 0

    lax.fori_loop(0, NUM_CHUNKS // 2, pair_step, 0)
    wait_scatter(NUM_CHUNKS - 2, buf0, ss0)
    wait_scatter(NUM_CHUNKS - 1, buf1, ss1)


@jax.jit
def kernel(X, table, bias):
    idx = X.reshape(NUM_WORKERS, GROUPS_PER_WORKER, GROUP)
    bias_tiled = jnp.tile(bias, (GROUP // N_FIELDS_K, 1))

    mesh = plsc.VectorSubcoreMesh(core_axis_name="c", subcore_axis_name="s")
    run = functools.partial(
        pl.kernel,
        mesh=mesh,
        out_type=jax.ShapeDtypeStruct((TOTAL_ROWS, EMBED_DIM_K), jnp.float32),
        scratch_types=[
            pltpu.VMEM((GROUPS_PER_WORKER, GROUP), jnp.int32),
            pltpu.VMEM((GROUP, EMBED_DIM_K), jnp.float32),
            pltpu.VMEM((CHUNK_ROWS, EMBED_DIM_K), jnp.float32),
            pltpu.VMEM((CHUNK_ROWS, EMBED_DIM_K), jnp.float32),
            pltpu.SemaphoreType.DMA,
            pltpu.SemaphoreType.DMA,
            pltpu.SemaphoreType.DMA,
            pltpu.SemaphoreType.DMA,
        ],
        compiler_params=pltpu.CompilerParams(use_tc_tiling_on_sc=False),
    )(_sc_body)
    out = run(table, idx, bias_tiled)
    return out.reshape(BATCH_K, N_FIELDS_K, EMBED_DIM_K)
